# baseline (speedup 1.0000x reference)
import functools

import jax
import jax.numpy as jnp
from jax import lax
from jax.experimental import pallas as pl
from jax.experimental.pallas import tpu as pltpu
from jax.experimental.pallas import tpu_sc as plsc

_B, _N, _D, _K = 4, 2048, 64, 20
_BN_SCALE = (1.0 + 1e-3) ** -0.5

_ROWS = 256
_NBLK = (_B * _N) // _ROWS
_KPAD = 32

_NW = 32
_PTS_W = (_B * _N) // _NW
_CH_P = 8
_CH_R = _CH_P * _K
_NCH = _PTS_W // _CH_P
_GR = 80
_IDXF = 176


def _topk_body(x_ref, w1_ref, b1_ref, w2_ref, b2_ref, w3_ref, b3_ref,
               wd_ref, bd_ref, idx_ref, xpad_ref):
    xb = x_ref[...]
    h = jnp.dot(xb, w1_ref[...], preferred_element_type=jnp.float32) + b1_ref[...]
    h = jnp.maximum(h, 0.0) * _BN_SCALE
    h = jnp.dot(h, w2_ref[...], preferred_element_type=jnp.float32) + b2_ref[...]
    h = jnp.maximum(h, 0.0) * _BN_SCALE
    h = jnp.dot(h, w3_ref[...], preferred_element_type=jnp.float32) + b3_ref[...]
    h = jnp.maximum(h, 0.0) * _BN_SCALE
    adj = jnp.dot(h, wd_ref[...], preferred_element_type=jnp.float32) + bd_ref[...]

    xpad_ref[...] = jnp.concatenate([xb, jnp.zeros((_ROWS, _D), jnp.float32)], axis=1)

    col = lax.broadcasted_iota(jnp.int32, (_ROWS, _N), 1).astype(jnp.float32)
    kcol = lax.broadcasted_iota(jnp.int32, (_ROWS, _KPAD), 1).astype(jnp.float32)
    base = pl.program_id(0) * _ROWS
    batch_off = (base // _N) * _N
    out = jnp.zeros((_ROWS, _KPAD), jnp.float32)
    for k in range(_K):
        m = jnp.min(adj, axis=1, keepdims=True)
        idx = jnp.min(jnp.where(adj == m, col, float(_N)), axis=1, keepdims=True)
        adj = jnp.where(col == idx, jnp.inf, adj)
        out = jnp.where(kcol == float(k), idx, out)
    idx_ref[...] = out.astype(jnp.int32) + batch_off


def _tc_topk(x_flat, w1, b1, w2, b2, w3, b3, wd, bd):
    full = lambda s: pl.BlockSpec(s, lambda i: (0, 0))
    return pl.pallas_call(
        _topk_body,
        grid=(_NBLK,),
        in_specs=[
            pl.BlockSpec((_ROWS, _D), lambda i: (i, 0)),
            full((_D, 32)), full((1, 32)),
            full((32, 64)), full((1, 64)),
            full((64, 512)), full((1, 512)),
            full((512, _N)), full((1, _N)),
        ],
        out_specs=[
            pl.BlockSpec((_ROWS, _KPAD), lambda i: (i, 0)),
            pl.BlockSpec((_ROWS, 2 * _D), lambda i: (i, 0)),
        ],
        out_shape=[
            jax.ShapeDtypeStruct((_B * _N, _KPAD), jnp.int32),
            jax.ShapeDtypeStruct((_B * _N, 2 * _D), jnp.float32),
        ],
    )(x_flat, w1, b1, w2, b2, w3, b3, wd, bd)


def _sc_edge_body(x_hbm, idx_hbm, out_hbm, idxp_v, idxf0, idxf1, neigh0,
                  neigh1, cent0, cent1, out0, out1, gsem0, gsem1, wsem0, wsem1):
    wid = lax.axis_index("s") * 2 + lax.axis_index("c")
    idxfs = (idxf0, idxf1)
    neighs = (neigh0, neigh1)
    cents = (cent0, cent1)
    outs = (out0, out1)
    gsems = (gsem0, gsem1)
    wsems = (wsem0, wsem1)

    pltpu.sync_copy(idx_hbm.at[pl.ds(wid * _PTS_W, _PTS_W)], idxp_v)

    def fire(c, b):
        for p in range(_CH_P):
            row = c * _CH_P + p
            idxfs[b][pl.ds(20 * p, 16)] = idxp_v[row, pl.ds(0, 16)]
            idxfs[b][pl.ds(20 * p + 16, 16)] = idxp_v[row, pl.ds(16, 16)]
        for s in range(_CH_R // _GR):
            pltpu.async_copy(x_hbm.at[idxfs[b].at[pl.ds(s * _GR, _GR)]],
                             neighs[b].at[pl.ds(s * _GR, _GR)], gsems[b])
        p0 = wid * _PTS_W + c * _CH_P
        pltpu.async_copy(x_hbm.at[pl.ds(p0, _CH_P)], cents[b], gsems[b])

    def wait_gather(b):
        pltpu.make_async_copy(x_hbm.at[pl.ds(0, _CH_R)],
                              neighs[b], gsems[b]).wait()
        pltpu.make_async_copy(x_hbm.at[pl.ds(0, _CH_P)],
                              cents[b], gsems[b]).wait()

    def wait_write(b):
        pltpu.make_async_copy(outs[b], out_hbm.at[pl.ds(0, _CH_R)],
                              wsems[b]).wait()

    fire(0, 0)

    def pair_body(i, carry):
        for b in range(2):
            c = 2 * i + b

            @pl.when(c + 1 < _NCH)
            def _prefetch():
                fire(c + 1, 1 - b)

            wait_gather(b)

            @pl.when(c >= 2)
            def _drain_write():
                wait_write(b)

            def pt_body(p, c2):
                for j in range(_D // 16):
                    cvec = cents[b][p, pl.ds(j * 16, 16)]
                    for k in range(_K):
                        r = p * _K + k
                        nvec = neighs[b][r, pl.ds(j * 16, 16)]
                        outs[b][r, pl.ds(j * 16, 16)] = cvec
                        outs[b][r, pl.ds(_D + j * 16, 16)] = nvec - cvec
                return c2

            lax.fori_loop(0, _CH_P, pt_body, 0)
            r0 = (wid * _PTS_W + c * _CH_P) * _K
            pltpu.async_copy(outs[b], out_hbm.at[pl.ds(r0, _CH_R)], wsems[b])
        return carry

    lax.fori_loop(0, _NCH // 2, pair_body, 0)
    wait_write(0)
    wait_write(1)


@functools.lru_cache(maxsize=1)
def _make_sc_edge():
    return functools.partial(
        pl.kernel,
        mesh=plsc.VectorSubcoreMesh(core_axis_name="c", subcore_axis_name="s"),
        out_type=jax.ShapeDtypeStruct((_B * _N * _K, 2 * _D), jnp.float32),
        scratch_types=[
            pltpu.VMEM((_PTS_W, _KPAD), jnp.int32),
            pltpu.VMEM((_IDXF,), jnp.int32),
            pltpu.VMEM((_IDXF,), jnp.int32),
            pltpu.VMEM((_CH_R, 2 * _D), jnp.float32),
            pltpu.VMEM((_CH_R, 2 * _D), jnp.float32),
            pltpu.VMEM((_CH_P, 2 * _D), jnp.float32),
            pltpu.VMEM((_CH_P, 2 * _D), jnp.float32),
            pltpu.VMEM((_CH_R, 2 * _D), jnp.float32),
            pltpu.VMEM((_CH_R, 2 * _D), jnp.float32),
            pltpu.SemaphoreType.DMA,
            pltpu.SemaphoreType.DMA,
            pltpu.SemaphoreType.DMA,
            pltpu.SemaphoreType.DMA,
        ],
    )(_sc_edge_body)


def kernel(x, W1, b1, W2, b2, W3, b3, Wd, bd):
    x_flat = x.reshape(_B * _N, _D)
    idx, x_pad = _tc_topk(x_flat, W1, b1.reshape(1, -1), W2, b2.reshape(1, -1),
                          W3, b3.reshape(1, -1), Wd, bd.reshape(1, -1))
    out = _make_sc_edge()(x_pad, idx)
    return out.reshape(_B, _N, _K, 2 * _D)


# speedup vs baseline: 1.2538x; 1.2538x over previous
import functools

import jax
import jax.numpy as jnp
from jax import lax
from jax.experimental import pallas as pl
from jax.experimental.pallas import tpu as pltpu
from jax.experimental.pallas import tpu_sc as plsc

_B, _N, _D, _K = 4, 2048, 64, 20
_BN_SCALE = (1.0 + 1e-3) ** -0.5

_ROWS = 256
_NBLK = (_B * _N) // _ROWS
_KPAD = 32

_NW = 32
_PTS_W = (_B * _N) // _NW
_CH_P = 8
_CH_R = _CH_P * _K
_NCH = _PTS_W // _CH_P
_GR = 80
_IDXF = 176


def _topk_body(x_ref, w1_ref, b1_ref, w2_ref, b2_ref, w3_ref, b3_ref,
               wd_ref, bd_ref, idx_ref, xpad_ref):
    xb = x_ref[...]
    h = jnp.dot(xb, w1_ref[...], preferred_element_type=jnp.float32) + b1_ref[...]
    h = jnp.maximum(h, 0.0) * _BN_SCALE
    h = jnp.dot(h, w2_ref[...], preferred_element_type=jnp.float32) + b2_ref[...]
    h = jnp.maximum(h, 0.0) * _BN_SCALE
    h = jnp.dot(h, w3_ref[...], preferred_element_type=jnp.float32) + b3_ref[...]
    h = jnp.maximum(h, 0.0) * _BN_SCALE
    adj = jnp.dot(h, wd_ref[...], preferred_element_type=jnp.float32) + bd_ref[...]

    xpad_ref[...] = jnp.concatenate([xb, jnp.zeros((_ROWS, _D), jnp.float32)], axis=1)

    col = lax.broadcasted_iota(jnp.int32, (_ROWS, _N), 1).astype(jnp.float32)
    kcol = lax.broadcasted_iota(jnp.int32, (_ROWS, _KPAD), 1).astype(jnp.float32)
    base = pl.program_id(0) * _ROWS
    batch_off = (base // _N) * _N
    out = jnp.zeros((_ROWS, _KPAD), jnp.float32)
    for k in range(_K):
        m = jnp.min(adj, axis=1, keepdims=True)
        idx = jnp.min(jnp.where(adj == m, col, float(_N)), axis=1, keepdims=True)
        adj = jnp.where(col == idx, jnp.inf, adj)
        out = jnp.where(kcol == float(k), idx, out)
    idx_ref[...] = out.astype(jnp.int32) + batch_off


def _tc_topk(x_flat, w1, b1, w2, b2, w3, b3, wd, bd):
    full = lambda s: pl.BlockSpec(s, lambda i: (0, 0))
    return pl.pallas_call(
        _topk_body,
        grid=(_NBLK,),
        in_specs=[
            pl.BlockSpec((_ROWS, _D), lambda i: (i, 0)),
            full((_D, 32)), full((1, 32)),
            full((32, 64)), full((1, 64)),
            full((64, 512)), full((1, 512)),
            full((512, _N)), full((1, _N)),
        ],
        out_specs=[
            pl.BlockSpec((_ROWS, _KPAD), lambda i: (i, 0)),
            pl.BlockSpec((_ROWS, 2 * _D), lambda i: (i, 0)),
        ],
        out_shape=[
            jax.ShapeDtypeStruct((_B * _N, _KPAD), jnp.int32),
            jax.ShapeDtypeStruct((_B * _N, 2 * _D), jnp.float32),
        ],
    )(x_flat, w1, b1, w2, b2, w3, b3, wd, bd)


def _sc_edge_body(x_hbm, idx_hbm, out_hbm, idxp_v, idxf0, idxf1, neigh0,
                  neigh1, cent0, cent1, out0, out1, gsem0, gsem1, wsem0, wsem1):
    wid = lax.axis_index("s") * 2 + lax.axis_index("c")
    idxfs = (idxf0, idxf1)
    neighs = (neigh0, neigh1)
    cents = (cent0, cent1)
    outs = (out0, out1)
    gsems = (gsem0, gsem1)
    wsems = (wsem0, wsem1)

    pltpu.sync_copy(idx_hbm.at[pl.ds(wid * _PTS_W, _PTS_W)], idxp_v)

    def fire(c, b):
        for p in range(_CH_P):
            row = c * _CH_P + p
            idxfs[b][pl.ds(20 * p, 16)] = idxp_v[row, pl.ds(0, 16)]
            idxfs[b][pl.ds(20 * p + 16, 16)] = idxp_v[row, pl.ds(16, 16)]
        for s in range(_CH_R // _GR):
            pltpu.async_copy(x_hbm.at[idxfs[b].at[pl.ds(s * _GR, _GR)]],
                             neighs[b].at[pl.ds(s * _GR, _GR)], gsems[b])
        p0 = wid * _PTS_W + c * _CH_P
        pltpu.async_copy(x_hbm.at[pl.ds(p0, _CH_P)], cents[b], gsems[b])

    def wait_gather(b):
        pltpu.make_async_copy(x_hbm.at[pl.ds(0, _CH_R)],
                              neighs[b], gsems[b]).wait()
        pltpu.make_async_copy(x_hbm.at[pl.ds(0, _CH_P)],
                              cents[b], gsems[b]).wait()

    def wait_write(b):
        pltpu.make_async_copy(outs[b], out_hbm.at[0, pl.ds(0, _CH_P)],
                              wsems[b]).wait()

    fire(0, 0)

    def pair_body(i, carry):
        for b in range(2):
            c = 2 * i + b

            @pl.when(c + 1 < _NCH)
            def _prefetch():
                fire(c + 1, 1 - b)

            wait_gather(b)

            @pl.when(c >= 2)
            def _drain_write():
                wait_write(b)

            def pt_body(p, c2):
                for j in range(_D // 16):
                    cvec = cents[b][p, pl.ds(j * 16, 16)]
                    for k in range(_K):
                        r = p * _K + k
                        nvec = neighs[b][r, pl.ds(j * 16, 16)]
                        outs[b][p, k, pl.ds(j * 16, 16)] = cvec
                        outs[b][p, k, pl.ds(_D + j * 16, 16)] = nvec - cvec
                return c2

            lax.fori_loop(0, _CH_P, pt_body, 0)
            b_ = wid // (_N // _PTS_W)
            n0 = (wid % (_N // _PTS_W)) * _PTS_W + c * _CH_P
            pltpu.async_copy(outs[b], out_hbm.at[b_, pl.ds(n0, _CH_P)], wsems[b])
        return carry

    lax.fori_loop(0, _NCH // 2, pair_body, 0)
    wait_write(0)
    wait_write(1)


@functools.lru_cache(maxsize=1)
def _make_sc_edge():
    return functools.partial(
        pl.kernel,
        mesh=plsc.VectorSubcoreMesh(core_axis_name="c", subcore_axis_name="s"),
        out_type=jax.ShapeDtypeStruct((_B, _N, _K, 2 * _D), jnp.float32),
        scratch_types=[
            pltpu.VMEM((_PTS_W, _KPAD), jnp.int32),
            pltpu.VMEM((_IDXF,), jnp.int32),
            pltpu.VMEM((_IDXF,), jnp.int32),
            pltpu.VMEM((_CH_R, 2 * _D), jnp.float32),
            pltpu.VMEM((_CH_R, 2 * _D), jnp.float32),
            pltpu.VMEM((_CH_P, 2 * _D), jnp.float32),
            pltpu.VMEM((_CH_P, 2 * _D), jnp.float32),
            pltpu.VMEM((_CH_P, _K, 2 * _D), jnp.float32),
            pltpu.VMEM((_CH_P, _K, 2 * _D), jnp.float32),
            pltpu.SemaphoreType.DMA,
            pltpu.SemaphoreType.DMA,
            pltpu.SemaphoreType.DMA,
            pltpu.SemaphoreType.DMA,
        ],
    )(_sc_edge_body)


def kernel(x, W1, b1, W2, b2, W3, b3, Wd, bd):
    x_flat = x.reshape(_B * _N, _D)
    idx, x_pad = _tc_topk(x_flat, W1, b1.reshape(1, -1), W2, b2.reshape(1, -1),
                          W3, b3.reshape(1, -1), Wd, bd.reshape(1, -1))
    return _make_sc_edge()(x_pad, idx)
